# baseline (speedup 1.0000x reference)
import jax
import jax.numpy as jnp
from jax import lax
from jax.experimental import pallas as pl
from jax.experimental.pallas import tpu as pltpu
from jax.experimental.pallas import tpu_sc as plsc

_NUM_FIELDS = 26
_VOCAB = 100000
_EMBED = 32
_BATCH = 16384

_NC, _NS, _LANES = 2, 16, 16
_NW = _NC * _NS
_BPW = _BATCH // _NW
_CHUNK = 128
_NCHUNK = _BPW // _CHUNK
_TCHUNK = _NUM_FIELDS * _NCHUNK
_GROUPS = _CHUNK // _LANES
_WROWS = _NUM_FIELDS * _VOCAB // 4


def _sc_body(gp_hbm, gc_hbm, w_hbm, out_hbm, gp_v, gc_v, buf_v, acc_v,
             sem0, sem1):
    sems = (sem0, sem1)
    wid = lax.axis_index("s") * _NC + lax.axis_index("c")
    base = wid * _BPW

    pltpu.sync_copy(gp_hbm.at[wid], gp_v)
    pltpu.sync_copy(gc_hbm.at[wid], gc_v)

    zeros = jnp.zeros((_LANES,), jnp.float32)

    def zrow(j, carry):
        def zcol(h, c2):
            acc_v[j, pl.ds(h * _LANES, _LANES)] = zeros
            return c2
        return lax.fori_loop(0, _BPW // _LANES, zcol, carry, unroll=8)
    lax.fori_loop(0, _EMBED, zrow, None)

    iotas = [lax.iota(jnp.int32, _LANES) + g * _LANES for g in range(_GROUPS)]

    def process(t, par):
        bloc = (t & (_NCHUNK - 1)) * _CHUNK
        for g in range(_GROUPS):
            colb = gc_v[t, pl.ds(g * _LANES, _LANES)]
            dst = bloc + g * _LANES
            for c in range(_EMBED):
                val = plsc.load_gather(buf_v.at[par], [iotas[g], colb + c])
                plsc.addupdate(acc_v.at[c, pl.ds(dst, _LANES)], val)

    def fire(t, par):
        return pltpu.async_copy(w_hbm.at[gp_v.at[t]], buf_v.at[par], sems[par])

    def drain(t, par):
        pltpu.make_async_copy(w_hbm.at[gp_v.at[t]], buf_v.at[par],
                              sems[par]).wait()

    fire(0, 0)

    def tbody(u, carry):
        t0 = u * 2
        fire(t0 + 1, 1)
        drain(t0, 0)
        process(t0, 0)
        @pl.when(t0 + 2 < _TCHUNK)
        def _():
            fire(t0 + 2, 0)
        drain(t0 + 1, 1)
        process(t0 + 1, 1)
        return carry

    lax.fori_loop(0, _TCHUNK // 2, tbody, None)

    pltpu.sync_copy(acc_v, out_hbm.at[:, pl.ds(base, _BPW)])


def kernel(x, W):
    offs = jnp.arange(_NUM_FIELDS, dtype=jnp.int32) * _VOCAB
    e = (x.T + offs[:, None])
    ew = e.reshape(_NUM_FIELDS, _NW, _NCHUNK, _CHUNK)
    ew = ew.transpose(1, 0, 2, 3).reshape(_NW, _TCHUNK, _CHUNK)
    gp = ew >> 2
    gc = (ew & 3) * _EMBED
    w128 = W.reshape(_WROWS, 4 * _EMBED)
    mesh = plsc.VectorSubcoreMesh(
        core_axis_name="c", subcore_axis_name="s",
        num_cores=_NC, num_subcores=_NS,
    )
    f = pl.kernel(
        _sc_body,
        out_type=jax.ShapeDtypeStruct((_EMBED, _BATCH), jnp.float32),
        mesh=mesh,
        scratch_types=[
            pltpu.VMEM((_TCHUNK, _CHUNK), jnp.int32),
            pltpu.VMEM((_TCHUNK, _CHUNK), jnp.int32),
            pltpu.VMEM((2, _CHUNK, 4 * _EMBED), jnp.float32),
            pltpu.VMEM((_EMBED, _BPW), jnp.float32),
            pltpu.SemaphoreType.DMA,
            pltpu.SemaphoreType.DMA,
        ],
        compiler_params=pltpu.CompilerParams(use_tc_tiling_on_sc=True,
                                            needs_layout_passes=False),
    )
    out_t = f(gp, gc, w128)
    return out_t.T
